# Initial kernel scaffold; baseline (speedup 1.0000x reference)
import jax
import jax.numpy as jnp
from jax.experimental import pallas as pl


def kernel(word_ids, char_ids, buffer_idx, stack_idx, legal_actions, word_table, char_table, W, b):
    raise NotImplementedError("write your pallas kernel here")


# speedup vs baseline: 18.2715x; 18.2715x over previous
import functools

import jax
import jax.numpy as jnp
from jax import lax
from jax.experimental import pallas as pl
from jax.experimental.pallas import tpu as pltpu
from jax.experimental.pallas import tpu_sc as plsc

B, L, C, H = 4096, 50, 6, 50
HP = 56
NC, NS = 2, 16
NW = NC * NS
ROWS_PER_W = B // NW
SUB = 16
SLOTS = SUB * 6
NCB = ROWS_PER_W // SUB


def _sc_gather_body(idx_hbm, wids_hbm, cids_hbm, wtab_hbm, ctab_hbm, out_hbm,
                    ids_w, ids_c, idx_v, widx, cidx, wrows, crows, acc, sem):
    w = lax.axis_index("s") * NC + lax.axis_index("c")

    def sub(cb, carry):
        row0 = w * ROWS_PER_W + cb * SUB
        slot0 = row0 * 6
        pltpu.sync_copy(idx_hbm.at[pl.ds(slot0, SLOTS)], idx_v)
        pltpu.sync_copy(wids_hbm.at[pl.ds(row0 * L, SUB * L)], ids_w)
        pltpu.sync_copy(cids_hbm.at[pl.ds(row0 * L * C, SUB * L * C)], ids_c)

        for g in range(SLOTS // 16):
            s = jnp.arange(16, dtype=jnp.int32) + g * 16
            i_vec = lax.div(s, jnp.int32(6))
            pos = idx_v[pl.ds(g * 16, 16)]
            tok = i_vec * L + pos
            wid_ids = plsc.load_gather(ids_w, [tok])
            widx[pl.ds(g * 16, 16)] = wid_ids
            cbase = tok * C
            for c in range(C):
                cid = plsc.load_gather(ids_c, [cbase + c])
                cidx[c, pl.ds(g * 16, 16)] = cid

        cps = [pltpu.async_copy(wtab_hbm.at[widx], wrows, sem)]
        for c in range(C):
            cps.append(pltpu.async_copy(ctab_hbm.at[cidx.at[c]], crows.at[c], sem))
        for cp in cps:
            cp.wait()

        def erow(r, carry2):
            for off in (0, 16, 32, 34):
                v = wrows[r, pl.ds(off, 16)]
                for c in range(C):
                    v = v + crows[c, r, pl.ds(off, 16)]
                acc[r, pl.ds(off, 16)] = v
            return carry2

        lax.fori_loop(0, SLOTS, erow, 0)
        pltpu.sync_copy(acc, out_hbm.at[pl.ds(slot0, SLOTS)])
        return carry

    lax.fori_loop(0, NCB, sub, 0)


@functools.partial(
    pl.kernel,
    out_type=jax.ShapeDtypeStruct((B * 6, H), jnp.float32),
    mesh=plsc.VectorSubcoreMesh(core_axis_name="c", subcore_axis_name="s"),
    scratch_types=[
        pltpu.VMEM((SUB * L,), jnp.int32),
        pltpu.VMEM((SUB * L * C,), jnp.int32),
        pltpu.VMEM((SLOTS,), jnp.int32),
        pltpu.VMEM((SLOTS,), jnp.int32),
        pltpu.VMEM((C, SLOTS), jnp.int32),
        pltpu.VMEM((SLOTS, HP), jnp.float32),
        pltpu.VMEM((C, SLOTS, HP), jnp.float32),
        pltpu.VMEM((SLOTS, H), jnp.float32),
        pltpu.SemaphoreType.DMA,
    ],
    compiler_params=pltpu.CompilerParams(
        use_tc_tiling_on_sc=False, needs_layout_passes=False),
)
def _sc_gather(*args):
    _sc_gather_body(*args)


def _tc_head_body(x_ref, w_ref, b_ref, legal_ref, o_ref):
    res = jnp.dot(x_ref[...], w_ref[...], preferred_element_type=jnp.float32)
    res = res + b_ref[...]
    res = jnp.clip(res, -1000000.0, 10.0)
    o_ref[...] = jnp.exp(res) * legal_ref[...]


def kernel(word_ids, char_ids, buffer_idx, stack_idx, legal_actions,
           word_table, char_table, W, b):
    idx_flat = jnp.concatenate([buffer_idx, stack_idx], axis=1).reshape(-1)
    wtab_p = jnp.pad(word_table, ((0, 0), (0, HP - H)))
    ctab_p = jnp.pad(char_table, ((0, 0), (0, HP - H)))
    rows = _sc_gather(idx_flat, word_ids.reshape(B * L),
                      char_ids.reshape(B * L * C), wtab_p, ctab_p)
    X = rows.reshape(B, 6 * H)
    out = pl.pallas_call(
        _tc_head_body,
        out_shape=jax.ShapeDtypeStruct((B, 3), jnp.float32),
    )(X, W, b.reshape(1, 3), legal_actions)
    return out


# speedup vs baseline: 43.0566x; 2.3565x over previous
import functools

import jax
import jax.numpy as jnp
from jax import lax
from jax.experimental import pallas as pl
from jax.experimental.pallas import tpu as pltpu
from jax.experimental.pallas import tpu_sc as plsc

B, L, C, H = 4096, 50, 6, 50
HPW = 128
HP = 56
NC, NS = 2, 16
NW = NC * NS
ROWS_PER_W = B // NW
SUB = 16
SLOTS = SUB * 6
NCB = ROWS_PER_W // SUB


def _sc_gather_body(idx_hbm, wids_hbm, cids_hbm, wtab_hbm, ctab_hbm, out_hbm,
                    ids_w0, ids_w1, ids_c0, ids_c1, idx_v0, idx_v1,
                    widx0, widx1, cidx0, cidx1, wrows0, wrows1,
                    crows0, crows1, acc0, acc1,
                    semS0, semS1, semG0, semG1):
    w = lax.axis_index("s") * NC + lax.axis_index("c")
    ids_w = (ids_w0, ids_w1)
    ids_c = (ids_c0, ids_c1)
    idx_v = (idx_v0, idx_v1)
    widx = (widx0, widx1)
    cidx = (cidx0, cidx1)
    wrows = (wrows0, wrows1)
    crows = (crows0, crows1)
    acc = (acc0, acc1)
    semS = (semS0, semS1)
    semG = (semG0, semG1)

    def stage(cb, p):
        row0 = w * ROWS_PER_W + cb * SUB
        return [
            pltpu.async_copy(idx_hbm.at[pl.ds(row0 * 6, SLOTS)], idx_v[p], semS[p]),
            pltpu.async_copy(wids_hbm.at[pl.ds(row0, SUB)], ids_w[p], semS[p]),
            pltpu.async_copy(cids_hbm.at[:, :, pl.ds(row0, SUB)], ids_c[p], semS[p]),
        ]

    def build(p):
        for g in range(SLOTS // 16):
            s = jnp.arange(16, dtype=jnp.int32) + g * 16
            i_vec = lax.div(s, jnp.int32(6))
            pos = idx_v[p][pl.ds(g * 16, 16)]
            wid_ids = plsc.load_gather(ids_w[p], [i_vec, pos])
            widx[p][pl.ds(g * 16, 16)] = wid_ids
            for c in range(C):
                cvec = jnp.full((16,), c, dtype=jnp.int32)
                cid = plsc.load_gather(ids_c[p], [cvec, pos, i_vec])
                cidx[p][c, pl.ds(g * 16, 16)] = cid

    def gathers(p):
        cps = [pltpu.async_copy(wtab_hbm.at[widx[p]], wrows[p], semG[p])]
        for c in range(C):
            cps.append(pltpu.async_copy(
                ctab_hbm.at[cidx[p].at[c]], crows[p].at[c], semG[p]))
        return cps

    def erow_out(cb, p):
        def erow(r, carry2):
            for off in (0, 16, 32, 34):
                v = wrows[p][r, pl.ds(off, 16)]
                for c in range(C):
                    v = v + crows[p][c, r, pl.ds(off, 16)]
                acc[p][r, pl.ds(off, 16)] = v
            return carry2

        lax.fori_loop(0, SLOTS, erow, 0)
        slot0 = (w * ROWS_PER_W + cb * SUB) * 6
        pltpu.sync_copy(acc[p], out_hbm.at[pl.ds(slot0, SLOTS)])

    hS = {0: stage(0, 0)}
    hG = {}
    for cb in range(NCB):
        p = cb & 1
        for h in hS[cb]:
            h.wait()
        build(p)
        hG[cb] = gathers(p)
        if cb + 1 < NCB:
            hS[cb + 1] = stage(cb + 1, (cb + 1) & 1)
        if cb > 0:
            erow_out(cb - 1, 1 - p)
        for h in hG[cb]:
            h.wait()
    erow_out(NCB - 1, (NCB - 1) & 1)


@functools.partial(
    pl.kernel,
    out_type=jax.ShapeDtypeStruct((B * 6, H), jnp.float32),
    mesh=plsc.VectorSubcoreMesh(core_axis_name="c", subcore_axis_name="s"),
    scratch_types=[
        pltpu.VMEM((SUB, L), jnp.int32),
        pltpu.VMEM((SUB, L), jnp.int32),
        pltpu.VMEM((C, L, SUB), jnp.int32),
        pltpu.VMEM((C, L, SUB), jnp.int32),
        pltpu.VMEM((SLOTS,), jnp.int32),
        pltpu.VMEM((SLOTS,), jnp.int32),
        pltpu.VMEM((SLOTS,), jnp.int32),
        pltpu.VMEM((SLOTS,), jnp.int32),
        pltpu.VMEM((C, SLOTS), jnp.int32),
        pltpu.VMEM((C, SLOTS), jnp.int32),
        pltpu.VMEM((SLOTS, HPW), jnp.float32),
        pltpu.VMEM((SLOTS, HPW), jnp.float32),
        pltpu.VMEM((C, SLOTS, HP), jnp.float32),
        pltpu.VMEM((C, SLOTS, HP), jnp.float32),
        pltpu.VMEM((SLOTS, H), jnp.float32),
        pltpu.VMEM((SLOTS, H), jnp.float32),
        pltpu.SemaphoreType.DMA,
        pltpu.SemaphoreType.DMA,
        pltpu.SemaphoreType.DMA,
        pltpu.SemaphoreType.DMA,
    ],
    compiler_params=pltpu.CompilerParams(
        use_tc_tiling_on_sc=False, needs_layout_passes=False),
)
def _sc_gather(*args):
    _sc_gather_body(*args)


def _tc_head_body(x_ref, w_ref, b_ref, legal_ref, o_ref):
    res = jnp.dot(x_ref[...], w_ref[...], preferred_element_type=jnp.float32)
    res = res + b_ref[...]
    res = jnp.clip(res, -1000000.0, 10.0)
    o_ref[...] = jnp.exp(res) * legal_ref[...]


def kernel(word_ids, char_ids, buffer_idx, stack_idx, legal_actions,
           word_table, char_table, W, b):
    idx_flat = jnp.concatenate([buffer_idx, stack_idx], axis=1).reshape(-1)
    wtab_p = jnp.pad(word_table, ((0, 0), (0, HPW - H)))
    ctab_p = jnp.pad(char_table.T, ((0, HP - H), (0, 0))).T
    cids_t = jnp.transpose(char_ids, (2, 1, 0))
    rows = _sc_gather(idx_flat, word_ids, cids_t, wtab_p, ctab_p)
    X = rows.reshape(B, 6 * H)
    out = pl.pallas_call(
        _tc_head_body,
        out_shape=jax.ShapeDtypeStruct((B, 3), jnp.float32),
    )(X, W, b.reshape(1, 3), legal_actions)
    return out
